# baseline (speedup 1.0000x reference)
import functools

import jax
import jax.numpy as jnp
from jax import lax
from jax.experimental import pallas as pl
from jax.experimental.pallas import tpu as pltpu
from jax.experimental.pallas import tpu_sc as plsc

EMBED = 64
CHUNK = 128
K = 4
LANES = 16
N_WORKERS = 32


@functools.lru_cache(maxsize=None)
def _build_transpose(embed: int, vocab: int):
    mesh = plsc.VectorSubcoreMesh(core_axis_name="c", subcore_axis_name="s")
    n_full = vocab // CHUNK
    rem = vocab % CHUNK
    out_rows = n_full * CHUNK + (CHUNK if rem else 0)

    @functools.partial(
        pl.kernel,
        mesh=mesh,
        out_type=jax.ShapeDtypeStruct((out_rows, 2 * EMBED), jnp.float32),
        scratch_types=[
            pltpu.VMEM((2, embed, CHUNK), jnp.float32),
            pltpu.VMEM((2, CHUNK, 2 * EMBED), jnp.float32),
            pltpu.SemaphoreType.DMA,
            pltpu.SemaphoreType.DMA,
        ],
        compiler_params=pltpu.CompilerParams(
            use_tc_tiling_on_sc=True, needs_layout_passes=False
        ),
    )
    def transpose_kernel(tt_hbm, tail_hbm, out_hbm, tile_v, outb_v, isem, osem):
        nc = lax.axis_size("c")
        wid = lax.axis_index("s") * nc + lax.axis_index("c")
        row_ids = [lax.iota(jnp.int32, LANES) + LANES * c
                   for c in range(CHUNK // LANES)]

        def in_desc(i, b):
            v0 = (wid + N_WORKERS * i) * CHUNK
            return pltpu.make_async_copy(
                tt_hbm.at[:, pl.ds(v0, CHUNK)], tile_v.at[b], isem
            )

        def out_desc(i, b):
            v0 = (wid + N_WORKERS * i) * CHUNK
            return pltpu.make_async_copy(
                outb_v.at[b], out_hbm.at[pl.ds(v0, CHUNK)], osem
            )

        n_mine = n_full // N_WORKERS + jnp.where(
            wid < n_full % N_WORKERS, 1, 0
        )
        pl.when(n_mine > 0)(lambda: in_desc(0, 0).start())

        def process(i, b):
            pl.when(i + 1 < n_mine)(lambda: in_desc(i + 1, 1 - b).start())
            in_desc(i, b).wait()
            pl.when(i >= 2)(lambda: out_desc(i - 2, b).wait())
            for e in range(embed):
                col = jnp.full((LANES,), e, jnp.int32)
                for c in range(CHUNK // LANES):
                    vec = tile_v[b, e, pl.ds(LANES * c, LANES)]
                    plsc.store_scatter(outb_v.at[b], [row_ids[c], col], vec)
            out_desc(i, b).start()

        def do_pair(i2, carry):
            for b in (0, 1):
                i = 2 * i2 + b
                pl.when(i < n_mine)(lambda i=i, b=b: process(i, b))
            return carry

        lax.fori_loop(0, (n_mine + 1) // 2, do_pair, 0)

        def drain(i):
            pl.when(i >= 0)(lambda: out_desc(i, lax.rem(i, 2)).wait())

        drain(n_mine - 2)
        drain(n_mine - 1)

        if rem:
            @pl.when(wid == N_WORKERS - 1)
            def _():
                pltpu.sync_copy(tail_hbm, outb_v.at[0])
                pltpu.sync_copy(
                    outb_v.at[0], out_hbm.at[pl.ds(n_full * CHUNK, CHUNK)]
                )

    return transpose_kernel


@functools.lru_cache(maxsize=None)
def _build_gather(seq: int, batch: int, table_rows: int):
    mesh = plsc.VectorSubcoreMesh(core_axis_name="c", subcore_axis_name="s")
    n_chunks = seq
    assert n_chunks % K == 0
    n_blocks = n_chunks // K
    blk = K * CHUNK

    @functools.partial(
        pl.kernel,
        mesh=mesh,
        out_type=jax.ShapeDtypeStruct((seq * batch, 2 * EMBED), jnp.float32),
        scratch_types=[
            pltpu.VMEM((n_chunks, CHUNK), jnp.int32),
            pltpu.VMEM((2, blk // 2, 2 * EMBED), jnp.float32),
            pltpu.SemaphoreType.DMA,
            pltpu.SemaphoreType.DMA,
        ],
        compiler_params=pltpu.CompilerParams(use_tc_tiling_on_sc=True),
    )
    def gather_kernel(table_hbm, idx_hbm, out_hbm, idx_v, rows_v, gsem, osem):
        nc = lax.axis_size("c")
        wid = lax.axis_index("s") * nc + lax.axis_index("c")
        col0 = wid * CHUNK
        pltpu.sync_copy(idx_hbm.at[:, pl.ds(col0, CHUNK)], idx_v)

        def fire(t, b):
            for k in range(K // 2):
                pltpu.make_async_copy(
                    table_hbm.at[idx_v.at[t * (K // 2) + k]],
                    rows_v.at[b, pl.ds(k * CHUNK, CHUNK)],
                    gsem,
                ).start()

        def wait_gathers(b):
            for k in range(K // 2):
                pltpu.make_async_copy(
                    table_hbm.at[idx_v.at[k]],
                    rows_v.at[b, pl.ds(k * CHUNK, CHUNK)],
                    gsem,
                ).wait()

        def out_copies(t, b):
            return [
                pltpu.make_async_copy(
                    rows_v.at[b, pl.ds(k * CHUNK, CHUNK)],
                    out_hbm.at[
                        pl.ds((t * (K // 2) + k) * batch + col0, CHUNK)
                    ],
                    osem,
                )
                for k in range(K // 2)
            ]

        n_b = n_chunks // (K // 2)
        fire(0, 0)

        def step(t, carry):
            b = lax.rem(t, 2)

            def drain_prev():
                for c in out_copies(t - 1, 1 - b):
                    c.wait()

            pl.when(t >= 1)(drain_prev)
            pl.when(t < n_b - 1)(lambda: fire(t + 1, 1 - b))
            wait_gathers(b)
            for c in out_copies(t, b):
                c.start()
            return carry

        lax.fori_loop(0, n_b, step, 0)
        for c in out_copies(n_b - 1, (n_b - 1) % 2):
            c.wait()

    return gather_kernel


def kernel(x, table):
    b, s = x.shape
    vocab, embed = table.shape
    assert embed == EMBED
    xt = jnp.swapaxes(x, 0, 1).astype(jnp.int32)
    tt = jnp.swapaxes(table, 0, 1)
    rem = vocab % CHUNK
    if rem:
        tail = jnp.pad(
            table[vocab - rem:, :], ((0, CHUNK - rem), (0, 2 * EMBED - embed))
        )
    else:
        tail = jnp.zeros((CHUNK, 2 * EMBED), jnp.float32)
    table128 = _build_transpose(embed, vocab)(tt, tail)
    out = _build_gather(s, b, table128.shape[0])(table128, xt)
    return jnp.swapaxes(out[:, :embed].reshape(s, b, embed), 0, 1)


# speedup vs baseline: 1.0107x; 1.0107x over previous
import functools

import jax
import jax.numpy as jnp
from jax import lax
from jax.experimental import pallas as pl
from jax.experimental.pallas import tpu as pltpu
from jax.experimental.pallas import tpu_sc as plsc

EMBED = 64
CHUNK = 128
K = 4
LANES = 16
N_WORKERS = 32


@functools.lru_cache(maxsize=None)
def _build_transpose(embed: int, vocab: int):
    mesh = plsc.VectorSubcoreMesh(core_axis_name="c", subcore_axis_name="s")
    n_full = vocab // CHUNK
    rem = vocab % CHUNK
    out_rows = n_full * CHUNK + (CHUNK if rem else 0)

    @functools.partial(
        pl.kernel,
        mesh=mesh,
        out_type=jax.ShapeDtypeStruct((out_rows * 2 * EMBED,), jnp.float32),
        scratch_types=[
            pltpu.VMEM((2, embed, CHUNK), jnp.float32),
            pltpu.VMEM((CHUNK * 2 * EMBED,), jnp.float32),
            pltpu.VMEM((CHUNK * 2 * EMBED,), jnp.float32),
            pltpu.SemaphoreType.DMA,
            pltpu.SemaphoreType.DMA,
        ],
        compiler_params=pltpu.CompilerParams(
            use_tc_tiling_on_sc=True, needs_layout_passes=False
        ),
    )
    def transpose_kernel(
        tt_hbm, tail_hbm, out_hbm, tile_v, outb0, outb1, isem, osem
    ):
        outb = (outb0, outb1)
        nc = lax.axis_size("c")
        wid = lax.axis_index("s") * nc + lax.axis_index("c")
        blk_words = CHUNK * 2 * EMBED
        base_ids = [
            (lax.iota(jnp.int32, LANES) + LANES * c) * (2 * EMBED)
            for c in range(CHUNK // LANES)
        ]

        def in_desc(i, b):
            v0 = (wid + N_WORKERS * i) * CHUNK
            return pltpu.make_async_copy(
                tt_hbm.at[:, pl.ds(v0, CHUNK)], tile_v.at[b], isem
            )

        def out_desc(i, b):
            w0 = (wid + N_WORKERS * i) * blk_words
            return pltpu.make_async_copy(
                outb[b], out_hbm.at[pl.ds(w0, blk_words)], osem
            )

        n_mine = n_full // N_WORKERS + jnp.where(
            wid < n_full % N_WORKERS, 1, 0
        )
        pl.when(n_mine > 0)(lambda: in_desc(0, 0).start())

        def process(i, b):
            pl.when(i + 1 < n_mine)(lambda: in_desc(i + 1, 1 - b).start())
            in_desc(i, b).wait()
            pl.when(i >= 2)(lambda: out_desc(i - 2, b).wait())
            def body_e(e, carry):
                vecs = [
                    tile_v[b, e, pl.ds(LANES * c, LANES)]
                    for c in range(CHUNK // LANES)
                ]
                idxs = [base + e for base in base_ids]
                for c in range(CHUNK // LANES):
                    plsc.store_scatter(outb[b], [idxs[c]], vecs[c])
                return carry

            lax.fori_loop(0, embed, body_e, 0)
            out_desc(i, b).start()

        def do_pair(i2, carry):
            for b in (0, 1):
                i = 2 * i2 + b
                pl.when(i < n_mine)(lambda i=i, b=b: process(i, b))
            return carry

        lax.fori_loop(0, (n_mine + 1) // 2, do_pair, 0)

        def drain(i):
            pl.when(i >= 0)(lambda: out_desc(0, 0).wait())

        drain(n_mine - 2)
        drain(n_mine - 1)

        if rem:
            @pl.when(wid == N_WORKERS - 1)
            def _():
                pltpu.sync_copy(tail_hbm, outb0)
                pltpu.sync_copy(
                    outb0, out_hbm.at[pl.ds(n_full * blk_words, blk_words)]
                )

    return transpose_kernel


@functools.lru_cache(maxsize=None)
def _build_gather(seq: int, batch: int, table_rows: int):
    mesh = plsc.VectorSubcoreMesh(core_axis_name="c", subcore_axis_name="s")
    n_chunks = seq
    assert n_chunks % K == 0
    n_blocks = n_chunks // K
    blk = K * CHUNK

    @functools.partial(
        pl.kernel,
        mesh=mesh,
        out_type=jax.ShapeDtypeStruct((seq * batch, 2 * EMBED), jnp.float32),
        scratch_types=[
            pltpu.VMEM((n_chunks, CHUNK), jnp.int32),
            pltpu.VMEM((2, blk // 2, 2 * EMBED), jnp.float32),
            pltpu.SemaphoreType.DMA,
            pltpu.SemaphoreType.DMA,
        ],
        compiler_params=pltpu.CompilerParams(use_tc_tiling_on_sc=True),
    )
    def gather_kernel(table_hbm, idx_hbm, out_hbm, idx_v, rows_v, gsem, osem):
        nc = lax.axis_size("c")
        wid = lax.axis_index("s") * nc + lax.axis_index("c")
        col0 = wid * CHUNK
        pltpu.sync_copy(idx_hbm.at[:, pl.ds(col0, CHUNK)], idx_v)

        def fire(t, b):
            for k in range(K // 2):
                pltpu.make_async_copy(
                    table_hbm.at[idx_v.at[t * (K // 2) + k]],
                    rows_v.at[b, pl.ds(k * CHUNK, CHUNK)],
                    gsem,
                ).start()

        def wait_gathers(b):
            for k in range(K // 2):
                pltpu.make_async_copy(
                    table_hbm.at[idx_v.at[k]],
                    rows_v.at[b, pl.ds(k * CHUNK, CHUNK)],
                    gsem,
                ).wait()

        def out_copies(t, b):
            return [
                pltpu.make_async_copy(
                    rows_v.at[b, pl.ds(k * CHUNK, CHUNK)],
                    out_hbm.at[
                        pl.ds((t * (K // 2) + k) * batch + col0, CHUNK)
                    ],
                    osem,
                )
                for k in range(K // 2)
            ]

        n_b = n_chunks // (K // 2)
        fire(0, 0)

        def step(t, carry):
            b = lax.rem(t, 2)

            def drain_prev():
                for c in out_copies(t - 1, 1 - b):
                    c.wait()

            pl.when(t >= 1)(drain_prev)
            pl.when(t < n_b - 1)(lambda: fire(t + 1, 1 - b))
            wait_gathers(b)
            for c in out_copies(t, b):
                c.start()
            return carry

        lax.fori_loop(0, n_b, step, 0)
        for c in out_copies(n_b - 1, (n_b - 1) % 2):
            c.wait()

    return gather_kernel


def kernel(x, table):
    b, s = x.shape
    vocab, embed = table.shape
    assert embed == EMBED
    xt = jnp.swapaxes(x, 0, 1).astype(jnp.int32)
    tt = jnp.swapaxes(table, 0, 1)
    rem = vocab % CHUNK
    if rem:
        tail = jnp.pad(
            table[vocab - rem:, :], ((0, CHUNK - rem), (0, 2 * EMBED - embed))
        )
    else:
        tail = jnp.zeros((CHUNK, 2 * EMBED), jnp.float32)
    tflat = _build_transpose(embed, vocab)(tt, tail.reshape(-1))
    table128 = tflat.reshape(-1, 2 * EMBED)
    out = _build_gather(s, b, table128.shape[0])(table128, xt)
    return jnp.swapaxes(out[:, :embed].reshape(s, b, embed), 0, 1)
